# Initial kernel scaffold; baseline (speedup 1.0000x reference)
import jax
import jax.numpy as jnp
from jax.experimental import pallas as pl


def kernel(x, edge_index, W, b, prelu_a):
    raise NotImplementedError("write your pallas kernel here")


# speedup vs baseline: 19.8832x; 19.8832x over previous
import functools

import jax
import jax.numpy as jnp
from jax import lax
from jax.experimental import pallas as pl
from jax.experimental.pallas import tpu as pltpu
from jax.experimental.pallas import tpu_sc as plsc

NC, NS = 2, 16
NT = NC * NS
K = 128


def _sc_mesh():
    return plsc.VectorSubcoreMesh(core_axis_name="c", subcore_axis_name="s")


def _make_deg(NP, C):
    RPT = NP // NS

    @functools.partial(
        pl.kernel,
        out_type=jax.ShapeDtypeStruct((NC, NP), jnp.float32),
        mesh=_sc_mesh(),
        scratch_types=[
            pltpu.VMEM((C, K), jnp.int32),
            pltpu.VMEM((K,), jnp.float32),
            pltpu.VMEM((RPT,), jnp.float32),
            pltpu.VMEM_SHARED((NP,), jnp.float32),
        ],
    )
    def deg_kernel(dst_hbm, ones_hbm, zeros_hbm, deg_hbm,
                   dst_v, ones_v, zeros_v, deg_sh):
        c = lax.axis_index("c")
        s = lax.axis_index("s")
        tid = s * NC + c
        pltpu.sync_copy(ones_hbm, ones_v)
        pltpu.sync_copy(zeros_hbm.at[pl.ds(0, RPT)], zeros_v)
        pltpu.sync_copy(zeros_v, deg_sh.at[pl.ds(s * RPT, RPT)])
        plsc.subcore_barrier()
        pltpu.sync_copy(dst_hbm.at[tid], dst_v)

        def body(j, carry):
            pltpu.sync_copy(ones_v, deg_sh.at[dst_v.at[j]], add=True)
            return carry

        lax.fori_loop(0, C, body, 0)
        plsc.subcore_barrier()
        pltpu.sync_copy(deg_sh.at[pl.ds(s * RPT, RPT)],
                        deg_hbm.at[c, pl.ds(s * RPT, RPT)])

    return deg_kernel


def _make_agg(NP, C, H):
    RPT = NP // NS
    NZ = RPT // K

    @functools.partial(
        pl.kernel,
        out_type=jax.ShapeDtypeStruct((NC, NP, H), jnp.float32),
        mesh=_sc_mesh(),
        scratch_types=[
            pltpu.VMEM((C, K), jnp.int32),
            pltpu.VMEM((C, K), jnp.int32),
            pltpu.VMEM((K, H), jnp.float32),
            pltpu.VMEM_SHARED((NP, H), jnp.float32),
            pltpu.SemaphoreType.DMA,
        ],
    )
    def agg_kernel(hs_hbm, src_hbm, dst_hbm, zeros_hbm, out_hbm,
                   src_v, dst_v, rows_v, agg_sh, sem):
        c = lax.axis_index("c")
        s = lax.axis_index("s")
        tid = s * NC + c
        pltpu.sync_copy(zeros_hbm, rows_v)
        for q in range(NZ):
            pltpu.sync_copy(rows_v, agg_sh.at[pl.ds(s * RPT + q * K, K)])
        plsc.subcore_barrier()
        pltpu.sync_copy(src_hbm.at[tid], src_v)
        pltpu.sync_copy(dst_hbm.at[tid], dst_v)

        def body(j, carry):
            pltpu.async_copy(hs_hbm.at[src_v.at[j]], rows_v, sem).wait()
            pltpu.sync_copy(rows_v, agg_sh.at[dst_v.at[j]], add=True)
            return carry

        lax.fori_loop(0, C, body, 0)
        plsc.subcore_barrier()
        for q in range(NZ):
            pltpu.sync_copy(agg_sh.at[pl.ds(s * RPT + q * K, K)],
                            out_hbm.at[c, pl.ds(s * RPT + q * K, K)])

    return agg_kernel


def _mm_body(x_ref, w_ref, deg_ref, hs_ref, dinv_ref):
    h = jnp.dot(x_ref[...], w_ref[...], preferred_element_type=jnp.float32)
    degsum = deg_ref[0] + deg_ref[1] + 1.0
    dinv = lax.rsqrt(degsum)
    r = lax.broadcasted_iota(jnp.int32, (128, 128), 0)
    f = lax.broadcasted_iota(jnp.int32, (128, 128), 1)
    eye = jnp.where(r == f, 1.0, 0.0).astype(jnp.float32)
    ones = jnp.ones((128, 128), jnp.float32)
    parts = []
    for q in range(dinv.shape[0]):
        diag = eye * dinv[q][None, :]
        parts.append(jnp.dot(diag, ones, preferred_element_type=jnp.float32))
    dinvb = jnp.concatenate(parts, axis=0)
    dinv_ref[...] = dinvb
    hs_ref[...] = h * dinvb


def _make_mm(NP, D, H, BM):
    QB = BM // 128
    return pl.pallas_call(
        _mm_body,
        grid=(NP // BM,),
        in_specs=[
            pl.BlockSpec((BM, D), lambda i: (i, 0)),
            pl.BlockSpec((D, H), lambda i: (0, 0)),
            pl.BlockSpec((NC, QB, 128), lambda i: (0, i, 0)),
        ],
        out_specs=[
            pl.BlockSpec((BM, H), lambda i: (i, 0)),
            pl.BlockSpec((BM, 128), lambda i: (i, 0)),
        ],
        out_shape=[
            jax.ShapeDtypeStruct((NP, H), jnp.float32),
            jax.ShapeDtypeStruct((NP, 128), jnp.float32),
        ],
    )


def _ep_body(agg_ref, hs_ref, dinv_ref, b_ref, a_ref, out_ref):
    z = dinv_ref[...] * (agg_ref[0] + agg_ref[1] + hs_ref[...]) + b_ref[...]
    out_ref[...] = jnp.where(z >= 0, z, a_ref[...] * z)


def _make_ep(NP, H, BM):
    return pl.pallas_call(
        _ep_body,
        grid=(NP // BM,),
        in_specs=[
            pl.BlockSpec((NC, BM, H), lambda i: (0, i, 0)),
            pl.BlockSpec((BM, H), lambda i: (i, 0)),
            pl.BlockSpec((BM, 128), lambda i: (i, 0)),
            pl.BlockSpec((1, H), lambda i: (0, 0)),
            pl.BlockSpec((1, H), lambda i: (0, 0)),
        ],
        out_specs=pl.BlockSpec((BM, H), lambda i: (i, 0)),
        out_shape=jax.ShapeDtypeStruct((NP, H), jnp.float32),
    )


def kernel(x, edge_index, W, b, prelu_a):
    N, D = x.shape
    H = W.shape[1]
    E = edge_index.shape[1]

    NP = (N // 2048 + 1) * 2048
    C = -(-E // (NT * K))
    Epad = NT * C * K
    pad = Epad - E

    srcp = jnp.concatenate(
        [edge_index[0], jnp.zeros((pad,), edge_index.dtype)]).reshape(NT, C, K)
    dstp = jnp.concatenate(
        [edge_index[1], jnp.full((pad,), N, edge_index.dtype)]).reshape(NT, C, K)
    xp = jnp.zeros((NP, D), x.dtype).at[:N, :].set(x)

    ones_k = jnp.ones((K,), jnp.float32)
    zeros_np = jnp.zeros((NP,), jnp.float32)
    zeros_kh = jnp.zeros((K, H), jnp.float32)

    deg_p = _make_deg(NP, C)(dstp, ones_k, zeros_np)
    hs, dinvb = _make_mm(NP, D, H, 1024)(
        xp, W, deg_p.reshape(NC, NP // 128, 128))
    agg_p = _make_agg(NP, C, H)(hs, srcp, dstp, zeros_kh)
    out = _make_ep(NP, H, 1024)(
        agg_p, hs, dinvb, b.reshape(1, H),
        jnp.broadcast_to(prelu_a.reshape(1, 1), (1, H)))
    return out[:N]


# speedup vs baseline: 35.8257x; 1.8018x over previous
import functools

import jax
import jax.numpy as jnp
from jax import lax
from jax.experimental import pallas as pl
from jax.experimental.pallas import tpu as pltpu
from jax.experimental.pallas import tpu_sc as plsc

NC, NS = 2, 16
NT = NC * NS
K = 128


def _sc_mesh():
    return plsc.VectorSubcoreMesh(core_axis_name="c", subcore_axis_name="s")


def _make_deg(NP, C):
    RPT = NP // NS

    @functools.partial(
        pl.kernel,
        out_type=jax.ShapeDtypeStruct((NC, NP), jnp.float32),
        mesh=_sc_mesh(),
        scratch_types=[
            pltpu.VMEM((C, K), jnp.int32),
            pltpu.VMEM((K,), jnp.float32),
            pltpu.VMEM((RPT,), jnp.float32),
            pltpu.VMEM_SHARED((NP,), jnp.float32),
            pltpu.SemaphoreType.DMA,
        ],
    )
    def deg_kernel(dst_hbm, ones_hbm, zeros_hbm, deg_hbm,
                   dst_v, ones_v, zeros_v, deg_sh, sem):
        c = lax.axis_index("c")
        s = lax.axis_index("s")
        tid = s * NC + c
        pltpu.sync_copy(ones_hbm, ones_v)
        pltpu.sync_copy(zeros_hbm.at[pl.ds(0, RPT)], zeros_v)
        pltpu.sync_copy(zeros_v, deg_sh.at[pl.ds(s * RPT, RPT)])
        plsc.subcore_barrier()
        pltpu.sync_copy(dst_hbm.at[tid], dst_v)

        def fire(j, carry):
            pltpu.async_copy(ones_v, deg_sh.at[dst_v.at[j]], sem, add=True)
            return carry

        lax.fori_loop(0, C, fire, 0)

        def drain(j, carry):
            pltpu.make_async_copy(
                deg_hbm.at[0, pl.ds(0, K)], ones_v, sem).wait()
            return carry

        lax.fori_loop(0, C, drain, 0)
        plsc.subcore_barrier()
        pltpu.sync_copy(deg_sh.at[pl.ds(s * RPT, RPT)],
                        deg_hbm.at[c, pl.ds(s * RPT, RPT)])

    return deg_kernel


def _make_agg(NP, C2, HH):
    RPT = NP // NS
    NZ = RPT // K
    NB = 6
    LA = 3

    @functools.partial(
        pl.kernel,
        out_type=jax.ShapeDtypeStruct((NC, NP, HH), jnp.float32),
        mesh=_sc_mesh(),
        compiler_params=pltpu.CompilerParams(use_tc_tiling_on_sc=False),
        scratch_types=[
            pltpu.VMEM((C2, K), jnp.int32),
            pltpu.VMEM((C2, K), jnp.int32),
            pltpu.VMEM((NB, K, HH), jnp.float32),
            pltpu.VMEM_SHARED((NP, HH), jnp.float32),
            pltpu.SemaphoreType.DMA((NB,)),
            pltpu.SemaphoreType.DMA((NB,)),
        ],
    )
    def agg_kernel(hs_hbm, src_hbm, dst_hbm, zeros_hbm, out_hbm,
                   src_v, dst_v, rows_v, agg_sh, gsem, ssem):
        c = lax.axis_index("c")
        s = lax.axis_index("s")
        pltpu.sync_copy(zeros_hbm, rows_v.at[0])
        for q in range(NZ):
            pltpu.sync_copy(rows_v.at[0], agg_sh.at[pl.ds(s * RPT + q * K, K)])
        plsc.subcore_barrier()
        pltpu.sync_copy(src_hbm.at[s], src_v)
        pltpu.sync_copy(dst_hbm.at[s], dst_v)
        half = hs_hbm.at[c]

        def gfire(j, b):
            pltpu.async_copy(half.at[src_v.at[j]], rows_v.at[b], gsem.at[b])

        def gwait(b):
            pltpu.make_async_copy(
                half.at[pl.ds(0, K)], rows_v.at[b], gsem.at[b]).wait()

        def sfire(j, b):
            pltpu.async_copy(rows_v.at[b], agg_sh.at[dst_v.at[j]],
                             ssem.at[b], add=True)

        def swait(b):
            pltpu.make_async_copy(
                half.at[pl.ds(0, K)], rows_v.at[b], ssem.at[b]).wait()

        for j in range(LA):
            gfire(j, j % NB)

        def body(j, carry):
            b = lax.rem(j, NB)
            bf = lax.rem(j + LA, NB)

            @pl.when(j + LA < C2)
            def _fire():
                @pl.when(j + LA >= NB)
                def _drain():
                    swait(bf)
                gfire(j + LA, bf)

            gwait(b)
            sfire(j, b)
            return carry

        lax.fori_loop(0, C2, body, 0)
        for b in range(NB):
            swait(b)
        plsc.subcore_barrier()
        for q in range(NZ):
            pltpu.sync_copy(agg_sh.at[pl.ds(s * RPT + q * K, K)],
                            out_hbm.at[c, pl.ds(s * RPT + q * K, K)])

    return agg_kernel


def _dinv_rows(deg_blk):
    degsum = deg_blk[0] + deg_blk[1] + 1.0
    dinv = lax.rsqrt(degsum)
    r = lax.broadcasted_iota(jnp.int32, (128, 128), 0)
    f = lax.broadcasted_iota(jnp.int32, (128, 128), 1)
    eye = jnp.where(r == f, 1.0, 0.0).astype(jnp.float32)
    ones = jnp.ones((128, 128), jnp.float32)
    parts = []
    for q in range(dinv.shape[0]):
        diag = eye * dinv[q][None, :]
        parts.append(jnp.dot(diag, ones, preferred_element_type=jnp.float32))
    return jnp.concatenate(parts, axis=0)


def _mm_body(x_ref, w_ref, deg_ref, hs_ref):
    h = jnp.dot(x_ref[...], w_ref[...], preferred_element_type=jnp.float32)
    hs = h * _dinv_rows(deg_ref)
    HH = hs.shape[1] // 2
    hs_ref[0] = hs[:, :HH]
    hs_ref[1] = hs[:, HH:]


def _make_mm(NP, D, H, BM):
    QB = BM // 128
    return pl.pallas_call(
        _mm_body,
        grid=(NP // BM,),
        in_specs=[
            pl.BlockSpec((BM, D), lambda i: (i, 0)),
            pl.BlockSpec((D, H), lambda i: (0, 0)),
            pl.BlockSpec((NC, QB, 128), lambda i: (0, i, 0)),
        ],
        out_specs=pl.BlockSpec((NC, BM, H // 2), lambda i: (0, i, 0)),
        out_shape=jax.ShapeDtypeStruct((NC, NP, H // 2), jnp.float32),
    )


def _ep_body(agg_ref, hs_ref, deg_ref, b_ref, a_ref, out_ref):
    agg = jnp.concatenate([agg_ref[0], agg_ref[1]], axis=1)
    hs = jnp.concatenate([hs_ref[0], hs_ref[1]], axis=1)
    z = _dinv_rows(deg_ref) * (agg + hs) + b_ref[...]
    out_ref[...] = jnp.where(z >= 0, z, a_ref[...] * z)


def _make_ep(NP, H, BM):
    QB = BM // 128
    return pl.pallas_call(
        _ep_body,
        grid=(NP // BM,),
        in_specs=[
            pl.BlockSpec((NC, BM, H // 2), lambda i: (0, i, 0)),
            pl.BlockSpec((NC, BM, H // 2), lambda i: (0, i, 0)),
            pl.BlockSpec((NC, QB, 128), lambda i: (0, i, 0)),
            pl.BlockSpec((1, H), lambda i: (0, 0)),
            pl.BlockSpec((1, H), lambda i: (0, 0)),
        ],
        out_specs=pl.BlockSpec((BM, H), lambda i: (i, 0)),
        out_shape=jax.ShapeDtypeStruct((NP, H), jnp.float32),
    )


def kernel(x, edge_index, W, b, prelu_a):
    N, D = x.shape
    H = W.shape[1]
    E = edge_index.shape[1]

    NP = (N // 2048 + 1) * 2048
    C = -(-E // (NT * K))
    C2 = -(-E // (NS * K))

    pad = NT * C * K - E
    dstp = jnp.concatenate(
        [edge_index[1], jnp.full((pad,), N, edge_index.dtype)]).reshape(NT, C, K)
    pad2 = NS * C2 * K - E
    srcp2 = jnp.concatenate(
        [edge_index[0], jnp.zeros((pad2,), edge_index.dtype)]).reshape(NS, C2, K)
    dstp2 = jnp.concatenate(
        [edge_index[1], jnp.full((pad2,), N, edge_index.dtype)]).reshape(NS, C2, K)
    xp = jnp.zeros((NP, D), x.dtype).at[:N, :].set(x)

    ones_k = jnp.ones((K,), jnp.float32)
    zeros_np = jnp.zeros((NP,), jnp.float32)
    zeros_kh = jnp.zeros((K, H // 2), jnp.float32)

    deg_p = _make_deg(NP, C)(dstp, ones_k, zeros_np)
    deg_r = deg_p.reshape(NC, NP // 128, 128)
    hs = _make_mm(NP, D, H, 1024)(xp, W, deg_r)
    agg = _make_agg(NP, C2, H // 2)(hs, srcp2, dstp2, zeros_kh)
    out = _make_ep(NP, H, 1024)(
        agg, hs, deg_r, b.reshape(1, H),
        jnp.broadcast_to(prelu_a.reshape(1, 1), (1, H)))
    return out[:N]


# speedup vs baseline: 40.2069x; 1.1223x over previous
import functools

import jax
import jax.numpy as jnp
from jax import lax
from jax.experimental import pallas as pl
from jax.experimental.pallas import tpu as pltpu
from jax.experimental.pallas import tpu_sc as plsc

NC, NS = 2, 16
NT = NC * NS
K = 128


def _sc_mesh():
    return plsc.VectorSubcoreMesh(core_axis_name="c", subcore_axis_name="s")


def _make_deg(NP, C):
    RPT = NP // NS

    @functools.partial(
        pl.kernel,
        out_type=jax.ShapeDtypeStruct((NC, NP), jnp.float32),
        mesh=_sc_mesh(),
        scratch_types=[
            pltpu.VMEM((C, K), jnp.int32),
            pltpu.VMEM((K,), jnp.float32),
            pltpu.VMEM((RPT,), jnp.float32),
            pltpu.VMEM_SHARED((NP,), jnp.float32),
            pltpu.SemaphoreType.DMA,
        ],
    )
    def deg_kernel(dst_hbm, ones_hbm, zeros_hbm, deg_hbm,
                   dst_v, ones_v, zeros_v, deg_sh, sem):
        c = lax.axis_index("c")
        s = lax.axis_index("s")
        tid = s * NC + c
        pltpu.sync_copy(ones_hbm, ones_v)
        pltpu.sync_copy(zeros_hbm.at[pl.ds(0, RPT)], zeros_v)
        pltpu.sync_copy(zeros_v, deg_sh.at[pl.ds(s * RPT, RPT)])
        plsc.subcore_barrier()
        pltpu.sync_copy(dst_hbm.at[tid], dst_v)

        def fire(j, carry):
            pltpu.async_copy(ones_v, deg_sh.at[dst_v.at[j]], sem, add=True)
            return carry

        lax.fori_loop(0, C, fire, 0)

        def drain(j, carry):
            pltpu.make_async_copy(
                deg_hbm.at[0, pl.ds(0, K)], ones_v, sem).wait()
            return carry

        lax.fori_loop(0, C, drain, 0)
        plsc.subcore_barrier()
        pltpu.sync_copy(deg_sh.at[pl.ds(s * RPT, RPT)],
                        deg_hbm.at[c, pl.ds(s * RPT, RPT)])

    return deg_kernel


def _make_agg(NP, C2, HH):
    RPT = NP // NS
    NZ = RPT // K
    NB = 8
    LA = 4

    @functools.partial(
        pl.kernel,
        out_type=jax.ShapeDtypeStruct((NC, NP, HH), jnp.bfloat16),
        mesh=_sc_mesh(),
        compiler_params=pltpu.CompilerParams(use_tc_tiling_on_sc=False),
        scratch_types=[
            pltpu.VMEM((C2, K), jnp.int32),
            pltpu.VMEM((C2, K), jnp.int32),
            pltpu.VMEM((NB, K, HH), jnp.bfloat16),
            pltpu.VMEM_SHARED((NP, HH), jnp.bfloat16),
            pltpu.SemaphoreType.DMA((NB,)),
            pltpu.SemaphoreType.DMA((NB,)),
        ],
    )
    def agg_kernel(hs_hbm, src_hbm, dst_hbm, zeros_hbm, out_hbm,
                   src_v, dst_v, rows_v, agg_sh, gsem, ssem):
        c = lax.axis_index("c")
        s = lax.axis_index("s")
        pltpu.sync_copy(zeros_hbm, rows_v.at[0])
        for q in range(NZ):
            pltpu.sync_copy(rows_v.at[0], agg_sh.at[pl.ds(s * RPT + q * K, K)])
        plsc.subcore_barrier()
        pltpu.sync_copy(src_hbm.at[s], src_v)
        pltpu.sync_copy(dst_hbm.at[s], dst_v)
        half = hs_hbm.at[c]

        def gfire(j, b):
            pltpu.async_copy(half.at[src_v.at[j]], rows_v.at[b], gsem.at[b])

        def gwait(b):
            pltpu.make_async_copy(
                half.at[pl.ds(0, K)], rows_v.at[b], gsem.at[b]).wait()

        def sfire(j, b):
            pltpu.async_copy(rows_v.at[b], agg_sh.at[dst_v.at[j]],
                             ssem.at[b], add=True)

        def swait(b):
            pltpu.make_async_copy(
                half.at[pl.ds(0, K)], rows_v.at[b], ssem.at[b]).wait()

        for j in range(LA):
            gfire(j, j % NB)

        def body(j, carry):
            b = lax.rem(j, NB)
            bf = lax.rem(j + LA, NB)

            @pl.when(j + LA < C2)
            def _fire():
                @pl.when(j + LA >= NB)
                def _drain():
                    swait(bf)
                gfire(j + LA, bf)

            gwait(b)
            sfire(j, b)
            return carry

        lax.fori_loop(0, C2, body, 0)
        for b in range(NB):
            swait(b)
        plsc.subcore_barrier()
        for q in range(NZ):
            pltpu.sync_copy(agg_sh.at[pl.ds(s * RPT + q * K, K)],
                            out_hbm.at[c, pl.ds(s * RPT + q * K, K)])

    return agg_kernel


def _dinv_rows(deg_blk):
    degsum = deg_blk[0] + deg_blk[1] + 1.0
    dinv = lax.rsqrt(degsum)
    r = lax.broadcasted_iota(jnp.int32, (128, 128), 0)
    f = lax.broadcasted_iota(jnp.int32, (128, 128), 1)
    eye = jnp.where(r == f, 1.0, 0.0).astype(jnp.float32)
    ones = jnp.ones((128, 128), jnp.float32)
    parts = []
    for q in range(dinv.shape[0]):
        diag = eye * dinv[q][None, :]
        parts.append(jnp.dot(diag, ones, preferred_element_type=jnp.float32))
    return jnp.concatenate(parts, axis=0)


def _mm_body(x_ref, w_ref, deg_ref, hs_ref):
    h = jnp.dot(x_ref[...], w_ref[...], preferred_element_type=jnp.float32)
    hs = (h * _dinv_rows(deg_ref)).astype(jnp.bfloat16)
    HH = hs.shape[1] // 2
    hs_ref[0] = hs[:, :HH]
    hs_ref[1] = hs[:, HH:]


def _make_mm(N, NP, D, H, BM):
    QB = BM // 128
    return pl.pallas_call(
        _mm_body,
        grid=(NP // BM,),
        in_specs=[
            pl.BlockSpec((BM, D), lambda i: (i, 0)),
            pl.BlockSpec((D, H), lambda i: (0, 0)),
            pl.BlockSpec((NC, QB, 128), lambda i: (0, i, 0)),
        ],
        out_specs=pl.BlockSpec((NC, BM, H // 2), lambda i: (0, i, 0)),
        out_shape=jax.ShapeDtypeStruct((NC, NP, H // 2), jnp.bfloat16),
    )


def _ep_body(agg_ref, hs_ref, deg_ref, b_ref, a_ref, out_ref):
    agg = jnp.concatenate([agg_ref[0], agg_ref[1]], axis=1).astype(jnp.float32)
    hs = jnp.concatenate([hs_ref[0], hs_ref[1]], axis=1).astype(jnp.float32)
    z = _dinv_rows(deg_ref) * (agg + hs) + b_ref[...]
    out_ref[...] = jnp.where(z >= 0, z, a_ref[...] * z)


def _make_ep(NP, H, BM):
    QB = BM // 128
    return pl.pallas_call(
        _ep_body,
        grid=(NP // BM,),
        in_specs=[
            pl.BlockSpec((NC, BM, H // 2), lambda i: (0, i, 0)),
            pl.BlockSpec((NC, BM, H // 2), lambda i: (0, i, 0)),
            pl.BlockSpec((NC, QB, 128), lambda i: (0, i, 0)),
            pl.BlockSpec((1, H), lambda i: (0, 0)),
            pl.BlockSpec((1, H), lambda i: (0, 0)),
        ],
        out_specs=pl.BlockSpec((BM, H), lambda i: (i, 0)),
        out_shape=jax.ShapeDtypeStruct((NP, H), jnp.float32),
    )


def kernel(x, edge_index, W, b, prelu_a):
    N, D = x.shape
    H = W.shape[1]
    E = edge_index.shape[1]

    NP = (N // 2048 + 1) * 2048
    C = -(-E // (NT * K))
    Epad = NT * C * K
    C2 = Epad // (NS * K)

    pad = Epad - E
    srcp = jnp.pad(edge_index[0], (0, pad))
    dstp = jnp.pad(edge_index[1], (0, pad), constant_values=N)

    ones_k = jnp.ones((K,), jnp.float32)
    zeros_np = jnp.zeros((NP,), jnp.float32)
    zeros_kh = jnp.zeros((K, H // 2), jnp.bfloat16)

    deg_p = _make_deg(NP, C)(dstp.reshape(NT, C, K), ones_k, zeros_np)
    deg_r = deg_p.reshape(NC, NP // 128, 128)
    hs = _make_mm(N, NP, D, H, 1024)(x, W, deg_r)
    agg = _make_agg(NP, C2, H // 2)(
        hs, srcp.reshape(NS, C2, K), dstp.reshape(NS, C2, K), zeros_kh)
    out = _make_ep(NP, H, 1024)(
        agg, hs, deg_r, b.reshape(1, H),
        jnp.broadcast_to(prelu_a.reshape(1, 1), (1, H)))
    return out[:N]
